# baseline (speedup 1.0000x reference)
import jax
import jax.numpy as jnp
from jax.experimental import pallas as pl
from jax.experimental.pallas import tpu as pltpu
from jax.experimental.pallas import tpu_sc as plsc

_B = 16
_N = 4096
_M = 2048
_D = 128


_CHUNK = 128


_G = 2
_GB = _B // _G


def _fps_body(pxyz_ref, idx_ref, px_ref, py_ref, pz_ref, dists_ref):
    lane_c = jax.lax.broadcasted_iota(jnp.int32, (_GB, _CHUNK), 1)

    dists_ref[...] = jnp.full((_B, _N), jnp.inf, dtype=jnp.float32)

    fzero = jnp.zeros((_GB, _CHUNK), jnp.float32)
    izero = jnp.zeros((_GB, _CHUNK), jnp.int32)

    n_slabs = _N // _CHUNK
    n_part = 2
    per_part = n_slabs // n_part

    lane_f = lane_c.astype(jnp.float32)
    ones_mm = jnp.ones((_CHUNK, _CHUNK), jnp.float32)

    def one_group(g, i_local, lx, ly, lz, ai, ax, ay, az):
        rows = slice(g * _GB, (g + 1) * _GB)
        offs = (
            jax.lax.broadcasted_iota(jnp.int32, (_GB, 1), 0) + g * _GB
        ) * _N
        partials = []
        for p in range(n_part):
            pm = am = cx = cy = cz = None
            for k in range(p * per_part, (p + 1) * per_part):
                sl = slice(k * _CHUNK, (k + 1) * _CHUNK)
                px = pxyz_ref[0, rows, sl]
                py = pxyz_ref[1, rows, sl]
                pz = pxyz_ref[2, rows, sl]
                dx = px - lx
                dy = py - ly
                dz = pz - lz
                d = dx * dx + dz * dz + dy * dy
                nd = jnp.minimum(dists_ref[rows, sl], d)
                dists_ref[rows, sl] = nd
                if pm is None:
                    pm = nd
                    am = jnp.full((_GB, _CHUNK), float(k), jnp.float32)
                    cx, cy, cz = px, py, pz
                else:
                    take = nd > pm
                    pm = jnp.where(take, nd, pm)
                    am = jnp.where(take, float(k), am)
                    cx = jnp.where(take, px, cx)
                    cy = jnp.where(take, py, cy)
                    cz = jnp.where(take, pz, cz)
            partials.append((pm, am, cx, cy, cz))

        take = partials[1][0] > partials[0][0]
        pm, am, cx, cy, cz = (
            jnp.where(take, vb, va)
            for va, vb in zip(partials[0], partials[1])
        )
        maxd = jnp.max(pm, axis=1, keepdims=True)
        cand = jnp.where(pm == maxd, am * float(_CHUNK) + lane_f, float(_N))
        nxtf = jnp.min(cand, axis=1, keepdims=True)
        oh = cand == nxtf
        sel3 = jnp.concatenate(
            [jnp.where(oh, cx, 0.0),
             jnp.where(oh, cy, 0.0),
             jnp.where(oh, cz, 0.0)], axis=0
        )
        summed = jax.lax.dot_general(
            sel3, ones_mm, (((1,), (0,)), ((), ())),
            precision=jax.lax.Precision.HIGHEST,
            preferred_element_type=jnp.float32,
        )
        nlx = summed[0 * _GB:1 * _GB]
        nly = summed[1 * _GB:2 * _GB]
        nlz = summed[2 * _GB:3 * _GB]
        sel = lane_c == i_local
        ai = jnp.where(sel, nxtf.astype(jnp.int32) + offs, ai)
        ax = jnp.where(sel, nlx, ax)
        ay = jnp.where(sel, nly, ay)
        az = jnp.where(sel, nlz, az)
        return (nlx, nly, nlz, ai, ax, ay, az)

    def body(i_local, carry):
        return tuple(
            one_group(g, i_local, *carry[g]) for g in range(_G)
        )

    state = []
    for g in range(_G):
        rows = slice(g * _GB, (g + 1) * _GB)
        offs = (
            jax.lax.broadcasted_iota(jnp.int32, (_GB, 1), 0) + g * _GB
        ) * _N
        lx0 = pxyz_ref[0, rows, 0:1]
        ly0 = pxyz_ref[1, rows, 0:1]
        lz0 = pxyz_ref[2, rows, 0:1]
        state.append((
            jnp.broadcast_to(lx0, (_GB, _CHUNK)),
            jnp.broadcast_to(ly0, (_GB, _CHUNK)),
            jnp.broadcast_to(lz0, (_GB, _CHUNK)),
            jnp.where(lane_c == 0, offs, izero),
            jnp.where(lane_c == 0, lx0, fzero),
            jnp.where(lane_c == 0, ly0, fzero),
            jnp.where(lane_c == 0, lz0, fzero),
        ))
    state = tuple(state)

    for j in range(_M // _CHUNK):
        start = 1 if j == 0 else 0
        state = jax.lax.fori_loop(start, _CHUNK, body, state)
        lo = j * _CHUNK
        new_state = []
        for g in range(_G):
            rows = slice(g * _GB, (g + 1) * _GB)
            lx, ly, lz, ai, ax, ay, az = state[g]
            idx_ref[rows, lo:lo + _CHUNK] = ai
            px_ref[rows, lo:lo + _CHUNK] = ax
            py_ref[rows, lo:lo + _CHUNK] = ay
            pz_ref[rows, lo:lo + _CHUNK] = az
            new_state.append((lx, ly, lz, izero, fzero, fzero, fzero))
        state = tuple(new_state)


def _fps_pallas(pxyz):
    out_shape = [
        jax.ShapeDtypeStruct((_B, _M), jnp.int32),
        jax.ShapeDtypeStruct((_B, _M), jnp.float32),
        jax.ShapeDtypeStruct((_B, _M), jnp.float32),
        jax.ShapeDtypeStruct((_B, _M), jnp.float32),
    ]
    return pl.pallas_call(
        _fps_body,
        out_shape=out_shape,
        scratch_shapes=[pltpu.VMEM((_B, _N), jnp.float32)],
    )(pxyz)


def _sc_gather_rows(x, idx2d):
    num_indices = idx2d.shape[1]
    window = 128
    mesh = plsc.VectorSubcoreMesh(core_axis_name="core", subcore_axis_name="subcore")

    @pl.kernel(
        out_type=jax.ShapeDtypeStruct((num_indices, _D), x.dtype),
        mesh=mesh,
    )
    def kern(x_hbm, i_hbm, o_hbm):
        def body(i_vmem, o_vmem):
            pltpu.sync_copy(x_hbm.at[i_vmem.at[0]], o_vmem)

        pltpu.emit_pipeline(
            body,
            grid=(num_indices // window,),
            in_specs=[pl.BlockSpec((1, window), index_map=lambda i: (0, i))],
            out_specs=[pl.BlockSpec((window, _D), index_map=lambda i: (i, 0))],
            core_axis_name=("core", "subcore"),
            dimension_semantics=(pltpu.PARALLEL,),
        )(i_hbm, o_hbm)

    return kern(x, idx2d)


def kernel(pos, x, batch_ids):
    pxyz = pos.reshape(_B, _N, 3).transpose(2, 0, 1)
    idx, px, py, pz = _fps_pallas(pxyz)
    gidx = idx.reshape(1, _B * _M)
    new_x = _sc_gather_rows(x, gidx)
    new_pos = jnp.stack([px, py, pz], axis=-1).reshape(_B * _M, 3)
    new_batch = batch_ids.reshape(_B, _N)[:, :_M].reshape(-1)
    return (new_x, new_pos, new_batch)


# speedup vs baseline: 1.2864x; 1.2864x over previous
import jax
import jax.numpy as jnp
from jax.experimental import pallas as pl
from jax.experimental.pallas import tpu as pltpu
from jax.experimental.pallas import tpu_sc as plsc

_B = 16
_N = 4096
_M = 2048
_D = 128


_CHUNK = 128


_G = 2
_GB = _B // _G


def _fps_body(pxyz_ref, idx_ref, px_ref, py_ref, pz_ref, dists_ref):
    lane_c = jax.lax.broadcasted_iota(jnp.int32, (_GB, _CHUNK), 1)

    dists_ref[...] = jnp.full((_B, _N), jnp.inf, dtype=jnp.float32)

    fzero = jnp.zeros((_GB, _CHUNK), jnp.float32)
    izero = jnp.zeros((_GB, _CHUNK), jnp.int32)

    n_slabs = _N // _CHUNK
    n_part = 2
    per_part = n_slabs // n_part

    lane_f = lane_c.astype(jnp.float32)
    ones_mm = jnp.ones((_CHUNK, _CHUNK), jnp.float32)

    def one_group(g, i_local, lx, ly, lz, ai, ax, ay, az):
        rows = slice(g * _GB, (g + 1) * _GB)
        offs = (
            jax.lax.broadcasted_iota(jnp.int32, (_GB, 1), 0) + g * _GB
        ) * _N
        partials = []
        for p in range(n_part):
            pm = am = cx = cy = cz = None
            for k in range(p * per_part, (p + 1) * per_part):
                sl = slice(k * _CHUNK, (k + 1) * _CHUNK)
                px = pxyz_ref[0, rows, sl]
                py = pxyz_ref[1, rows, sl]
                pz = pxyz_ref[2, rows, sl]
                dx = px - lx
                dy = py - ly
                dz = pz - lz
                d = dx * dx + dz * dz + dy * dy
                nd = jnp.minimum(dists_ref[rows, sl], d)
                dists_ref[rows, sl] = nd
                if pm is None:
                    pm = nd
                    am = jnp.full((_GB, _CHUNK), float(k), jnp.float32)
                    cx, cy, cz = px, py, pz
                else:
                    take = nd > pm
                    pm = jnp.where(take, nd, pm)
                    am = jnp.where(take, float(k), am)
                    cx = jnp.where(take, px, cx)
                    cy = jnp.where(take, py, cy)
                    cz = jnp.where(take, pz, cz)
            partials.append((pm, am, cx, cy, cz))

        take = partials[1][0] > partials[0][0]
        pm, am, cx, cy, cz = (
            jnp.where(take, vb, va)
            for va, vb in zip(partials[0], partials[1])
        )
        maxd = jnp.max(pm, axis=1, keepdims=True)
        cand = jnp.where(pm == maxd, am * float(_CHUNK) + lane_f, float(_N))
        nxtf = jnp.min(cand, axis=1, keepdims=True)
        oh = cand == nxtf
        nlx = jnp.broadcast_to(
            jnp.sum(jnp.where(oh, cx, 0.0), axis=1, keepdims=True),
            (_GB, _CHUNK))
        nly = jnp.broadcast_to(
            jnp.sum(jnp.where(oh, cy, 0.0), axis=1, keepdims=True),
            (_GB, _CHUNK))
        nlz = jnp.broadcast_to(
            jnp.sum(jnp.where(oh, cz, 0.0), axis=1, keepdims=True),
            (_GB, _CHUNK))
        sel = lane_c == i_local
        ai = jnp.where(sel, nxtf.astype(jnp.int32) + offs, ai)
        ax = jnp.where(sel, nlx, ax)
        ay = jnp.where(sel, nly, ay)
        az = jnp.where(sel, nlz, az)
        return (nlx, nly, nlz, ai, ax, ay, az)

    def body(i_local, carry):
        return tuple(
            one_group(g, i_local, *carry[g]) for g in range(_G)
        )

    state = []
    for g in range(_G):
        rows = slice(g * _GB, (g + 1) * _GB)
        offs = (
            jax.lax.broadcasted_iota(jnp.int32, (_GB, 1), 0) + g * _GB
        ) * _N
        lx0 = pxyz_ref[0, rows, 0:1]
        ly0 = pxyz_ref[1, rows, 0:1]
        lz0 = pxyz_ref[2, rows, 0:1]
        state.append((
            jnp.broadcast_to(lx0, (_GB, _CHUNK)),
            jnp.broadcast_to(ly0, (_GB, _CHUNK)),
            jnp.broadcast_to(lz0, (_GB, _CHUNK)),
            jnp.where(lane_c == 0, offs, izero),
            jnp.where(lane_c == 0, lx0, fzero),
            jnp.where(lane_c == 0, ly0, fzero),
            jnp.where(lane_c == 0, lz0, fzero),
        ))
    state = tuple(state)

    for j in range(_M // _CHUNK):
        start = 1 if j == 0 else 0
        state = jax.lax.fori_loop(start, _CHUNK, body, state)
        lo = j * _CHUNK
        new_state = []
        for g in range(_G):
            rows = slice(g * _GB, (g + 1) * _GB)
            lx, ly, lz, ai, ax, ay, az = state[g]
            idx_ref[rows, lo:lo + _CHUNK] = ai
            px_ref[rows, lo:lo + _CHUNK] = ax
            py_ref[rows, lo:lo + _CHUNK] = ay
            pz_ref[rows, lo:lo + _CHUNK] = az
            new_state.append((lx, ly, lz, izero, fzero, fzero, fzero))
        state = tuple(new_state)


def _fps_pallas(pxyz):
    out_shape = [
        jax.ShapeDtypeStruct((_B, _M), jnp.int32),
        jax.ShapeDtypeStruct((_B, _M), jnp.float32),
        jax.ShapeDtypeStruct((_B, _M), jnp.float32),
        jax.ShapeDtypeStruct((_B, _M), jnp.float32),
    ]
    return pl.pallas_call(
        _fps_body,
        out_shape=out_shape,
        scratch_shapes=[pltpu.VMEM((_B, _N), jnp.float32)],
    )(pxyz)


def _sc_gather_rows(x, idx2d):
    num_indices = idx2d.shape[1]
    window = 128
    mesh = plsc.VectorSubcoreMesh(core_axis_name="core", subcore_axis_name="subcore")

    @pl.kernel(
        out_type=jax.ShapeDtypeStruct((num_indices, _D), x.dtype),
        mesh=mesh,
    )
    def kern(x_hbm, i_hbm, o_hbm):
        def body(i_vmem, o_vmem):
            pltpu.sync_copy(x_hbm.at[i_vmem.at[0]], o_vmem)

        pltpu.emit_pipeline(
            body,
            grid=(num_indices // window,),
            in_specs=[pl.BlockSpec((1, window), index_map=lambda i: (0, i))],
            out_specs=[pl.BlockSpec((window, _D), index_map=lambda i: (i, 0))],
            core_axis_name=("core", "subcore"),
            dimension_semantics=(pltpu.PARALLEL,),
        )(i_hbm, o_hbm)

    return kern(x, idx2d)


def kernel(pos, x, batch_ids):
    pxyz = pos.reshape(_B, _N, 3).transpose(2, 0, 1)
    idx, px, py, pz = _fps_pallas(pxyz)
    gidx = idx.reshape(1, _B * _M)
    new_x = _sc_gather_rows(x, gidx)
    new_pos = jnp.stack([px, py, pz], axis=-1).reshape(_B * _M, 3)
    new_batch = batch_ids.reshape(_B, _N)[:, :_M].reshape(-1)
    return (new_x, new_pos, new_batch)


# speedup vs baseline: 1.5900x; 1.2360x over previous
import jax
import jax.numpy as jnp
from jax.experimental import pallas as pl
from jax.experimental.pallas import tpu as pltpu
from jax.experimental.pallas import tpu_sc as plsc

_B = 16
_N = 4096
_M = 2048
_D = 128


_CHUNK = 128


_G = 2
_GB = _B // _G


def _fps_body(pxyz_ref, idx_ref, px_ref, py_ref, pz_ref, dists_ref):
    lane_c = jax.lax.broadcasted_iota(jnp.int32, (_GB, _CHUNK), 1)

    dists_ref[...] = jnp.full((_B, _N), jnp.inf, dtype=jnp.float32)

    fzero = jnp.zeros((_GB, _CHUNK), jnp.float32)
    izero = jnp.zeros((_GB, _CHUNK), jnp.int32)

    n_slabs = _N // _CHUNK
    n_part = 2
    per_part = n_slabs // n_part

    lane_f = lane_c.astype(jnp.float32)
    ones_mm = jnp.ones((_CHUNK, _CHUNK), jnp.float32)

    def one_group(g, i_local, lx, ly, lz, ai, ax, ay, az):
        rows = slice(g * _GB, (g + 1) * _GB)
        offs = (
            jax.lax.broadcasted_iota(jnp.int32, (_GB, 1), 0) + g * _GB
        ) * _N
        partials = []
        for p in range(n_part):
            pm = am = cx = cy = cz = None
            for k in range(p * per_part, (p + 1) * per_part):
                sl = slice(k * _CHUNK, (k + 1) * _CHUNK)
                px = pxyz_ref[0, rows, sl]
                py = pxyz_ref[1, rows, sl]
                pz = pxyz_ref[2, rows, sl]
                dx = px - lx
                dy = py - ly
                dz = pz - lz
                d = dx * dx + dz * dz + dy * dy
                nd = jnp.minimum(dists_ref[rows, sl], d)
                dists_ref[rows, sl] = nd
                if pm is None:
                    pm = nd
                    am = jnp.full((_GB, _CHUNK), float(k), jnp.float32)
                    cx, cy, cz = px, py, pz
                else:
                    take = nd > pm
                    pm = jnp.where(take, nd, pm)
                    am = jnp.where(take, float(k), am)
                    cx = jnp.where(take, px, cx)
                    cy = jnp.where(take, py, cy)
                    cz = jnp.where(take, pz, cz)
            partials.append((pm, am, cx, cy, cz))

        take = partials[1][0] > partials[0][0]
        pm, am, cx, cy, cz = (
            jnp.where(take, vb, va)
            for va, vb in zip(partials[0], partials[1])
        )
        lidx = jnp.argmax(pm, axis=1, keepdims=True)
        oh = lane_c == lidx
        amsel = jnp.sum(jnp.where(oh, am, 0.0), axis=1, keepdims=True)
        nxtf = lidx * n_slabs + amsel.astype(jnp.int32)
        nlx = jnp.broadcast_to(
            jnp.sum(jnp.where(oh, cx, 0.0), axis=1, keepdims=True),
            (_GB, _CHUNK))
        nly = jnp.broadcast_to(
            jnp.sum(jnp.where(oh, cy, 0.0), axis=1, keepdims=True),
            (_GB, _CHUNK))
        nlz = jnp.broadcast_to(
            jnp.sum(jnp.where(oh, cz, 0.0), axis=1, keepdims=True),
            (_GB, _CHUNK))
        sel = lane_c == i_local
        ai = jnp.where(sel, nxtf + offs, ai)
        ax = jnp.where(sel, nlx, ax)
        ay = jnp.where(sel, nly, ay)
        az = jnp.where(sel, nlz, az)
        return (nlx, nly, nlz, ai, ax, ay, az)

    def body(i_local, carry):
        return tuple(
            one_group(g, i_local, *carry[g]) for g in range(_G)
        )

    state = []
    for g in range(_G):
        rows = slice(g * _GB, (g + 1) * _GB)
        offs = (
            jax.lax.broadcasted_iota(jnp.int32, (_GB, 1), 0) + g * _GB
        ) * _N
        lx0 = pxyz_ref[0, rows, 0:1]
        ly0 = pxyz_ref[1, rows, 0:1]
        lz0 = pxyz_ref[2, rows, 0:1]
        state.append((
            jnp.broadcast_to(lx0, (_GB, _CHUNK)),
            jnp.broadcast_to(ly0, (_GB, _CHUNK)),
            jnp.broadcast_to(lz0, (_GB, _CHUNK)),
            jnp.where(lane_c == 0, offs, izero),
            jnp.where(lane_c == 0, lx0, fzero),
            jnp.where(lane_c == 0, ly0, fzero),
            jnp.where(lane_c == 0, lz0, fzero),
        ))
    state = tuple(state)

    for j in range(_M // _CHUNK):
        start = 1 if j == 0 else 0
        state = jax.lax.fori_loop(start, _CHUNK, body, state)
        lo = j * _CHUNK
        new_state = []
        for g in range(_G):
            rows = slice(g * _GB, (g + 1) * _GB)
            lx, ly, lz, ai, ax, ay, az = state[g]
            idx_ref[rows, lo:lo + _CHUNK] = ai
            px_ref[rows, lo:lo + _CHUNK] = ax
            py_ref[rows, lo:lo + _CHUNK] = ay
            pz_ref[rows, lo:lo + _CHUNK] = az
            new_state.append((lx, ly, lz, izero, fzero, fzero, fzero))
        state = tuple(new_state)


def _fps_pallas(pxyz):
    out_shape = [
        jax.ShapeDtypeStruct((_B, _M), jnp.int32),
        jax.ShapeDtypeStruct((_B, _M), jnp.float32),
        jax.ShapeDtypeStruct((_B, _M), jnp.float32),
        jax.ShapeDtypeStruct((_B, _M), jnp.float32),
    ]
    return pl.pallas_call(
        _fps_body,
        out_shape=out_shape,
        scratch_shapes=[pltpu.VMEM((_B, _N), jnp.float32)],
    )(pxyz)


def _sc_gather_rows(x, idx2d):
    num_indices = idx2d.shape[1]
    window = 128
    mesh = plsc.VectorSubcoreMesh(core_axis_name="core", subcore_axis_name="subcore")

    @pl.kernel(
        out_type=jax.ShapeDtypeStruct((num_indices, _D), x.dtype),
        mesh=mesh,
    )
    def kern(x_hbm, i_hbm, o_hbm):
        def body(i_vmem, o_vmem):
            pltpu.sync_copy(x_hbm.at[i_vmem.at[0]], o_vmem)

        pltpu.emit_pipeline(
            body,
            grid=(num_indices // window,),
            in_specs=[pl.BlockSpec((1, window), index_map=lambda i: (0, i))],
            out_specs=[pl.BlockSpec((window, _D), index_map=lambda i: (i, 0))],
            core_axis_name=("core", "subcore"),
            dimension_semantics=(pltpu.PARALLEL,),
        )(i_hbm, o_hbm)

    return kern(x, idx2d)


def kernel(pos, x, batch_ids):
    pxyz = pos.reshape(_B, _CHUNK, _N // _CHUNK, 3).transpose(3, 0, 2, 1)
    pxyz = pxyz.reshape(3, _B, _N)
    idx, px, py, pz = _fps_pallas(pxyz)
    gidx = idx.reshape(1, _B * _M)
    new_x = _sc_gather_rows(x, gidx)
    new_pos = jnp.stack([px, py, pz], axis=-1).reshape(_B * _M, 3)
    new_batch = batch_ids.reshape(_B, _N)[:, :_M].reshape(-1)
    return (new_x, new_pos, new_batch)


# speedup vs baseline: 1.7074x; 1.0738x over previous
import jax
import jax.numpy as jnp
from jax.experimental import pallas as pl
from jax.experimental.pallas import tpu as pltpu
from jax.experimental.pallas import tpu_sc as plsc

_B = 16
_N = 4096
_M = 2048
_D = 128


_CHUNK = 128


_G = 2
_GB = _B // _G


def _fps_body(pxyz_ref, idx_ref, px_ref, py_ref, pz_ref, dists_ref):
    lane_c = jax.lax.broadcasted_iota(jnp.int32, (_GB, _CHUNK), 1)

    dists_ref[...] = jnp.full((_B, _N), jnp.inf, dtype=jnp.float32)

    fzero = jnp.zeros((_GB, _CHUNK), jnp.float32)
    izero = jnp.zeros((_GB, _CHUNK), jnp.int32)

    n_slabs = _N // _CHUNK
    n_part = 2
    per_part = n_slabs // n_part

    lane_f = lane_c.astype(jnp.float32)
    ones_mm = jnp.ones((_CHUNK, _CHUNK), jnp.float32)

    def one_group(g, i_local, lx, ly, lz, ai, ax, ay, az):
        rows = slice(g * _GB, (g + 1) * _GB)
        offs = (
            jax.lax.broadcasted_iota(jnp.int32, (_GB, 1), 0) + g * _GB
        ) * _N
        partials = []
        for p in range(n_part):
            pm = am = cx = cy = cz = None
            for k in range(p * per_part, (p + 1) * per_part):
                sl = slice(k * _CHUNK, (k + 1) * _CHUNK)
                px = pxyz_ref[0, rows, sl]
                py = pxyz_ref[1, rows, sl]
                pz = pxyz_ref[2, rows, sl]
                dx = px - lx
                dy = py - ly
                dz = pz - lz
                d = dx * dx + dz * dz + dy * dy
                nd = jnp.minimum(dists_ref[rows, sl], d)
                dists_ref[rows, sl] = nd
                if pm is None:
                    pm = nd
                    am = jnp.full((_GB, _CHUNK), float(k), jnp.float32)
                    cx, cy, cz = px, py, pz
                else:
                    take = nd > pm
                    pm = jnp.where(take, nd, pm)
                    am = jnp.where(take, float(k), am)
                    cx = jnp.where(take, px, cx)
                    cy = jnp.where(take, py, cy)
                    cz = jnp.where(take, pz, cz)
            partials.append((pm, am, cx, cy, cz))

        take = partials[1][0] > partials[0][0]
        pm, am, cx, cy, cz = (
            jnp.where(take, vb, va)
            for va, vb in zip(partials[0], partials[1])
        )
        lidx = jnp.argmax(pm, axis=1, keepdims=True)
        oh = lane_c == lidx
        amsel = jnp.sum(jnp.where(oh, am, 0.0), axis=1, keepdims=True)
        nxtf = lidx * n_slabs + amsel.astype(jnp.int32)
        nlx = jnp.broadcast_to(
            jnp.sum(jnp.where(oh, cx, 0.0), axis=1, keepdims=True),
            (_GB, _CHUNK))
        nly = jnp.broadcast_to(
            jnp.sum(jnp.where(oh, cy, 0.0), axis=1, keepdims=True),
            (_GB, _CHUNK))
        nlz = jnp.broadcast_to(
            jnp.sum(jnp.where(oh, cz, 0.0), axis=1, keepdims=True),
            (_GB, _CHUNK))
        sel = lane_c == i_local
        ai = jnp.where(sel, nxtf + offs, ai)
        ax = jnp.where(sel, nlx, ax)
        ay = jnp.where(sel, nly, ay)
        az = jnp.where(sel, nlz, az)
        return (nlx, nly, nlz, ai, ax, ay, az)

    def body(i_local, carry):
        return tuple(
            one_group(g, i_local, *carry[g]) for g in range(_G)
        )

    state = []
    for g in range(_G):
        rows = slice(g * _GB, (g + 1) * _GB)
        offs = (
            jax.lax.broadcasted_iota(jnp.int32, (_GB, 1), 0) + g * _GB
        ) * _N
        lx0 = pxyz_ref[0, rows, 0:1]
        ly0 = pxyz_ref[1, rows, 0:1]
        lz0 = pxyz_ref[2, rows, 0:1]
        state.append((
            jnp.broadcast_to(lx0, (_GB, _CHUNK)),
            jnp.broadcast_to(ly0, (_GB, _CHUNK)),
            jnp.broadcast_to(lz0, (_GB, _CHUNK)),
            jnp.where(lane_c == 0, offs, izero),
            jnp.where(lane_c == 0, lx0, fzero),
            jnp.where(lane_c == 0, ly0, fzero),
            jnp.where(lane_c == 0, lz0, fzero),
        ))
    state = tuple(state)

    for j in range(_M // _CHUNK):
        start = 1 if j == 0 else 0
        state = jax.lax.fori_loop(start, _CHUNK, body, state, unroll=2)
        lo = j * _CHUNK
        new_state = []
        for g in range(_G):
            rows = slice(g * _GB, (g + 1) * _GB)
            lx, ly, lz, ai, ax, ay, az = state[g]
            idx_ref[rows, lo:lo + _CHUNK] = ai
            px_ref[rows, lo:lo + _CHUNK] = ax
            py_ref[rows, lo:lo + _CHUNK] = ay
            pz_ref[rows, lo:lo + _CHUNK] = az
            new_state.append((lx, ly, lz, izero, fzero, fzero, fzero))
        state = tuple(new_state)


def _fps_pallas(pxyz):
    out_shape = [
        jax.ShapeDtypeStruct((_B, _M), jnp.int32),
        jax.ShapeDtypeStruct((_B, _M), jnp.float32),
        jax.ShapeDtypeStruct((_B, _M), jnp.float32),
        jax.ShapeDtypeStruct((_B, _M), jnp.float32),
    ]
    return pl.pallas_call(
        _fps_body,
        out_shape=out_shape,
        scratch_shapes=[pltpu.VMEM((_B, _N), jnp.float32)],
    )(pxyz)


def _sc_gather_rows(x, idx2d):
    num_indices = idx2d.shape[1]
    window = 128
    mesh = plsc.VectorSubcoreMesh(core_axis_name="core", subcore_axis_name="subcore")

    @pl.kernel(
        out_type=jax.ShapeDtypeStruct((num_indices, _D), x.dtype),
        mesh=mesh,
    )
    def kern(x_hbm, i_hbm, o_hbm):
        def body(i_vmem, o_vmem):
            pltpu.sync_copy(x_hbm.at[i_vmem.at[0]], o_vmem)

        pltpu.emit_pipeline(
            body,
            grid=(num_indices // window,),
            in_specs=[pl.BlockSpec((1, window), index_map=lambda i: (0, i))],
            out_specs=[pl.BlockSpec((window, _D), index_map=lambda i: (i, 0))],
            core_axis_name=("core", "subcore"),
            dimension_semantics=(pltpu.PARALLEL,),
        )(i_hbm, o_hbm)

    return kern(x, idx2d)


def kernel(pos, x, batch_ids):
    pxyz = pos.reshape(_B, _CHUNK, _N // _CHUNK, 3).transpose(3, 0, 2, 1)
    pxyz = pxyz.reshape(3, _B, _N)
    idx, px, py, pz = _fps_pallas(pxyz)
    gidx = idx.reshape(1, _B * _M)
    new_x = _sc_gather_rows(x, gidx)
    new_pos = jnp.stack([px, py, pz], axis=-1).reshape(_B * _M, 3)
    new_batch = batch_ids.reshape(_B, _N)[:, :_M].reshape(-1)
    return (new_x, new_pos, new_batch)


# speedup vs baseline: 1.7565x; 1.0288x over previous
import jax
import jax.numpy as jnp
from jax.experimental import pallas as pl
from jax.experimental.pallas import tpu as pltpu
from jax.experimental.pallas import tpu_sc as plsc

_B = 16
_N = 4096
_M = 2048
_D = 128


_CHUNK = 128


_G = 2
_GB = _B // _G


def _fps_body(pxyz_ref, idx_ref, px_ref, py_ref, pz_ref, dists_ref):
    lane_c = jax.lax.broadcasted_iota(jnp.int32, (_GB, _CHUNK), 1)

    dists_ref[...] = jnp.full((_B, _N), jnp.inf, dtype=jnp.float32)

    fzero = jnp.zeros((_GB, _CHUNK), jnp.float32)
    izero = jnp.zeros((_GB, _CHUNK), jnp.int32)

    n_slabs = _N // _CHUNK
    n_part = 2
    per_part = n_slabs // n_part

    lane_f = lane_c.astype(jnp.float32)
    ones_mm = jnp.ones((_CHUNK, _CHUNK), jnp.float32)

    def one_group(g, i_local, lx, ly, lz, ai, ax, ay, az):
        rows = slice(g * _GB, (g + 1) * _GB)
        offs = (
            jax.lax.broadcasted_iota(jnp.int32, (_GB, 1), 0) + g * _GB
        ) * _N
        partials = []
        for p in range(n_part):
            pm = am = cx = cy = cz = None
            for k in range(p * per_part, (p + 1) * per_part):
                sl = slice(k * _CHUNK, (k + 1) * _CHUNK)
                px = pxyz_ref[0, rows, sl]
                py = pxyz_ref[1, rows, sl]
                pz = pxyz_ref[2, rows, sl]
                dx = px - lx
                dy = py - ly
                dz = pz - lz
                d = dx * dx + dz * dz + dy * dy
                nd = jnp.minimum(dists_ref[rows, sl], d)
                dists_ref[rows, sl] = nd
                if pm is None:
                    pm = nd
                    am = jnp.full((_GB, _CHUNK), float(k), jnp.float32)
                    cx, cy, cz = px, py, pz
                else:
                    take = nd > pm
                    pm = jnp.where(take, nd, pm)
                    am = jnp.where(take, float(k), am)
                    cx = jnp.where(take, px, cx)
                    cy = jnp.where(take, py, cy)
                    cz = jnp.where(take, pz, cz)
            partials.append((pm, am, cx, cy, cz))

        take = partials[1][0] > partials[0][0]
        pm, am, cx, cy, cz = (
            jnp.where(take, vb, va)
            for va, vb in zip(partials[0], partials[1])
        )
        lidx = jnp.argmax(pm, axis=1, keepdims=True)
        oh = lane_c == lidx
        amsel = jnp.sum(jnp.where(oh, am, 0.0), axis=1, keepdims=True)
        nxtf = lidx * n_slabs + amsel.astype(jnp.int32)
        nlx = jnp.broadcast_to(
            jnp.sum(jnp.where(oh, cx, 0.0), axis=1, keepdims=True),
            (_GB, _CHUNK))
        nly = jnp.broadcast_to(
            jnp.sum(jnp.where(oh, cy, 0.0), axis=1, keepdims=True),
            (_GB, _CHUNK))
        nlz = jnp.broadcast_to(
            jnp.sum(jnp.where(oh, cz, 0.0), axis=1, keepdims=True),
            (_GB, _CHUNK))
        sel = lane_c == i_local
        ai = jnp.where(sel, nxtf + offs, ai)
        ax = jnp.where(sel, nlx, ax)
        ay = jnp.where(sel, nly, ay)
        az = jnp.where(sel, nlz, az)
        return (nlx, nly, nlz, ai, ax, ay, az)

    def body(i_local, carry):
        return tuple(
            one_group(g, i_local, *carry[g]) for g in range(_G)
        )

    state = []
    for g in range(_G):
        rows = slice(g * _GB, (g + 1) * _GB)
        offs = (
            jax.lax.broadcasted_iota(jnp.int32, (_GB, 1), 0) + g * _GB
        ) * _N
        lx0 = pxyz_ref[0, rows, 0:1]
        ly0 = pxyz_ref[1, rows, 0:1]
        lz0 = pxyz_ref[2, rows, 0:1]
        state.append((
            jnp.broadcast_to(lx0, (_GB, _CHUNK)),
            jnp.broadcast_to(ly0, (_GB, _CHUNK)),
            jnp.broadcast_to(lz0, (_GB, _CHUNK)),
            jnp.where(lane_c == 0, offs, izero),
            jnp.where(lane_c == 0, lx0, fzero),
            jnp.where(lane_c == 0, ly0, fzero),
            jnp.where(lane_c == 0, lz0, fzero),
        ))
    state = tuple(state)

    for j in range(_M // _CHUNK):
        start = 1 if j == 0 else 0
        state = jax.lax.fori_loop(start, _CHUNK, body, state, unroll=4)
        lo = j * _CHUNK
        new_state = []
        for g in range(_G):
            rows = slice(g * _GB, (g + 1) * _GB)
            lx, ly, lz, ai, ax, ay, az = state[g]
            idx_ref[rows, lo:lo + _CHUNK] = ai
            px_ref[rows, lo:lo + _CHUNK] = ax
            py_ref[rows, lo:lo + _CHUNK] = ay
            pz_ref[rows, lo:lo + _CHUNK] = az
            new_state.append((lx, ly, lz, izero, fzero, fzero, fzero))
        state = tuple(new_state)


def _fps_pallas(pxyz):
    out_shape = [
        jax.ShapeDtypeStruct((_B, _M), jnp.int32),
        jax.ShapeDtypeStruct((_B, _M), jnp.float32),
        jax.ShapeDtypeStruct((_B, _M), jnp.float32),
        jax.ShapeDtypeStruct((_B, _M), jnp.float32),
    ]
    return pl.pallas_call(
        _fps_body,
        out_shape=out_shape,
        scratch_shapes=[pltpu.VMEM((_B, _N), jnp.float32)],
    )(pxyz)


def _sc_gather_rows(x, idx2d):
    num_indices = idx2d.shape[1]
    window = 128
    mesh = plsc.VectorSubcoreMesh(core_axis_name="core", subcore_axis_name="subcore")

    @pl.kernel(
        out_type=jax.ShapeDtypeStruct((num_indices, _D), x.dtype),
        mesh=mesh,
    )
    def kern(x_hbm, i_hbm, o_hbm):
        def body(i_vmem, o_vmem):
            pltpu.sync_copy(x_hbm.at[i_vmem.at[0]], o_vmem)

        pltpu.emit_pipeline(
            body,
            grid=(num_indices // window,),
            in_specs=[pl.BlockSpec((1, window), index_map=lambda i: (0, i))],
            out_specs=[pl.BlockSpec((window, _D), index_map=lambda i: (i, 0))],
            core_axis_name=("core", "subcore"),
            dimension_semantics=(pltpu.PARALLEL,),
        )(i_hbm, o_hbm)

    return kern(x, idx2d)


def kernel(pos, x, batch_ids):
    pxyz = pos.reshape(_B, _CHUNK, _N // _CHUNK, 3).transpose(3, 0, 2, 1)
    pxyz = pxyz.reshape(3, _B, _N)
    idx, px, py, pz = _fps_pallas(pxyz)
    gidx = idx.reshape(1, _B * _M)
    new_x = _sc_gather_rows(x, gidx)
    new_pos = jnp.stack([px, py, pz], axis=-1).reshape(_B * _M, 3)
    new_batch = batch_ids.reshape(_B, _N)[:, :_M].reshape(-1)
    return (new_x, new_pos, new_batch)


# speedup vs baseline: 1.7614x; 1.0028x over previous
import jax
import jax.numpy as jnp
from jax.experimental import pallas as pl
from jax.experimental.pallas import tpu as pltpu
from jax.experimental.pallas import tpu_sc as plsc

_B = 16
_N = 4096
_M = 2048
_D = 128


_CHUNK = 128


_G = 2
_GB = _B // _G


def _fps_body(pxyz_ref, idx_ref, px_ref, py_ref, pz_ref, dists_ref):
    lane_c = jax.lax.broadcasted_iota(jnp.int32, (_GB, _CHUNK), 1)

    dists_ref[...] = jnp.full((_B, _N), jnp.inf, dtype=jnp.float32)

    fzero = jnp.zeros((_GB, _CHUNK), jnp.float32)
    izero = jnp.zeros((_GB, _CHUNK), jnp.int32)

    n_slabs = _N // _CHUNK
    n_part = 2
    per_part = n_slabs // n_part

    lane_f = lane_c.astype(jnp.float32)
    ones_mm = jnp.ones((_CHUNK, _CHUNK), jnp.float32)

    def one_group(g, i_local, lx, ly, lz, ai, ax, ay, az):
        rows = slice(g * _GB, (g + 1) * _GB)
        offs = (
            jax.lax.broadcasted_iota(jnp.int32, (_GB, 1), 0) + g * _GB
        ) * _N
        partials = []
        for p in range(n_part):
            pm = am = cx = cy = cz = None
            for k in range(p * per_part, (p + 1) * per_part):
                sl = slice(k * _CHUNK, (k + 1) * _CHUNK)
                px = pxyz_ref[0, rows, sl]
                py = pxyz_ref[1, rows, sl]
                pz = pxyz_ref[2, rows, sl]
                dx = px - lx
                dy = py - ly
                dz = pz - lz
                d = dx * dx + dz * dz + dy * dy
                nd = jnp.minimum(dists_ref[rows, sl], d)
                dists_ref[rows, sl] = nd
                if pm is None:
                    pm = nd
                    am = jnp.full((_GB, _CHUNK), float(k), jnp.float32)
                    cx, cy, cz = px, py, pz
                else:
                    take = nd > pm
                    pm = jnp.where(take, nd, pm)
                    am = jnp.where(take, float(k), am)
                    cx = jnp.where(take, px, cx)
                    cy = jnp.where(take, py, cy)
                    cz = jnp.where(take, pz, cz)
            partials.append((pm, am, cx, cy, cz))

        take = partials[1][0] > partials[0][0]
        pm, am, cx, cy, cz = (
            jnp.where(take, vb, va)
            for va, vb in zip(partials[0], partials[1])
        )
        lidx = jnp.argmax(pm, axis=1, keepdims=True)
        oh = lane_c == lidx
        amsel = jnp.sum(jnp.where(oh, am, 0.0), axis=1, keepdims=True)
        nxtf = lidx * n_slabs + amsel.astype(jnp.int32)
        nlx = jnp.broadcast_to(
            jnp.sum(jnp.where(oh, cx, 0.0), axis=1, keepdims=True),
            (_GB, _CHUNK))
        nly = jnp.broadcast_to(
            jnp.sum(jnp.where(oh, cy, 0.0), axis=1, keepdims=True),
            (_GB, _CHUNK))
        nlz = jnp.broadcast_to(
            jnp.sum(jnp.where(oh, cz, 0.0), axis=1, keepdims=True),
            (_GB, _CHUNK))
        sel = lane_c == i_local
        ai = jnp.where(sel, nxtf + offs, ai)
        ax = jnp.where(sel, nlx, ax)
        ay = jnp.where(sel, nly, ay)
        az = jnp.where(sel, nlz, az)
        return (nlx, nly, nlz, ai, ax, ay, az)

    def body(i_local, carry):
        return tuple(
            one_group(g, i_local, *carry[g]) for g in range(_G)
        )

    state = []
    for g in range(_G):
        rows = slice(g * _GB, (g + 1) * _GB)
        offs = (
            jax.lax.broadcasted_iota(jnp.int32, (_GB, 1), 0) + g * _GB
        ) * _N
        lx0 = pxyz_ref[0, rows, 0:1]
        ly0 = pxyz_ref[1, rows, 0:1]
        lz0 = pxyz_ref[2, rows, 0:1]
        state.append((
            jnp.broadcast_to(lx0, (_GB, _CHUNK)),
            jnp.broadcast_to(ly0, (_GB, _CHUNK)),
            jnp.broadcast_to(lz0, (_GB, _CHUNK)),
            jnp.where(lane_c == 0, offs, izero),
            jnp.where(lane_c == 0, lx0, fzero),
            jnp.where(lane_c == 0, ly0, fzero),
            jnp.where(lane_c == 0, lz0, fzero),
        ))
    state = tuple(state)

    for j in range(_M // _CHUNK):
        start = 1 if j == 0 else 0
        state = jax.lax.fori_loop(start, _CHUNK, body, state, unroll=8)
        lo = j * _CHUNK
        new_state = []
        for g in range(_G):
            rows = slice(g * _GB, (g + 1) * _GB)
            lx, ly, lz, ai, ax, ay, az = state[g]
            idx_ref[rows, lo:lo + _CHUNK] = ai
            px_ref[rows, lo:lo + _CHUNK] = ax
            py_ref[rows, lo:lo + _CHUNK] = ay
            pz_ref[rows, lo:lo + _CHUNK] = az
            new_state.append((lx, ly, lz, izero, fzero, fzero, fzero))
        state = tuple(new_state)


def _fps_pallas(pxyz):
    out_shape = [
        jax.ShapeDtypeStruct((_B, _M), jnp.int32),
        jax.ShapeDtypeStruct((_B, _M), jnp.float32),
        jax.ShapeDtypeStruct((_B, _M), jnp.float32),
        jax.ShapeDtypeStruct((_B, _M), jnp.float32),
    ]
    return pl.pallas_call(
        _fps_body,
        out_shape=out_shape,
        scratch_shapes=[pltpu.VMEM((_B, _N), jnp.float32)],
    )(pxyz)


def _sc_gather_rows(x, idx2d):
    num_indices = idx2d.shape[1]
    window = 128
    mesh = plsc.VectorSubcoreMesh(core_axis_name="core", subcore_axis_name="subcore")

    @pl.kernel(
        out_type=jax.ShapeDtypeStruct((num_indices, _D), x.dtype),
        mesh=mesh,
    )
    def kern(x_hbm, i_hbm, o_hbm):
        def body(i_vmem, o_vmem):
            pltpu.sync_copy(x_hbm.at[i_vmem.at[0]], o_vmem)

        pltpu.emit_pipeline(
            body,
            grid=(num_indices // window,),
            in_specs=[pl.BlockSpec((1, window), index_map=lambda i: (0, i))],
            out_specs=[pl.BlockSpec((window, _D), index_map=lambda i: (i, 0))],
            core_axis_name=("core", "subcore"),
            dimension_semantics=(pltpu.PARALLEL,),
        )(i_hbm, o_hbm)

    return kern(x, idx2d)


def kernel(pos, x, batch_ids):
    pxyz = pos.reshape(_B, _CHUNK, _N // _CHUNK, 3).transpose(3, 0, 2, 1)
    pxyz = pxyz.reshape(3, _B, _N)
    idx, px, py, pz = _fps_pallas(pxyz)
    gidx = idx.reshape(1, _B * _M)
    new_x = _sc_gather_rows(x, gidx)
    new_pos = jnp.stack([px, py, pz], axis=-1).reshape(_B * _M, 3)
    new_batch = batch_ids.reshape(_B, _N)[:, :_M].reshape(-1)
    return (new_x, new_pos, new_batch)
